# Initial kernel scaffold; baseline (speedup 1.0000x reference)
import functools

import jax
import jax.numpy as jnp
from jax import lax
from jax.experimental import pallas as pl
from jax.experimental.pallas import tpu as pltpu
from jax.experimental.pallas import tpu_sc as plsc

NUM_CORES = 2
NUM_SUBCORES = 16
NUM_WORKERS = NUM_CORES * NUM_SUBCORES
LANES = 16

IDX_W = 128
CHUNK = 1664
GRP = CHUNK // IDX_W
PERIOD = 208


@functools.partial(jax.jit, static_argnums=(3, 4))
def _gather_flat(tab, idx2, off, n_rows, d):
    per_w = n_rows // NUM_WORKERS
    n_chunks = per_w // CHUNK
    rows_per_w = per_w // IDX_W
    mesh = plsc.VectorSubcoreMesh(core_axis_name="c", subcore_axis_name="s")

    @functools.partial(
        pl.kernel,
        out_type=jax.ShapeDtypeStruct((n_rows, d), jnp.float32),
        mesh=mesh,
        scratch_types=[
            pltpu.VMEM((GRP, IDX_W), jnp.int32),
            pltpu.VMEM((CHUNK, d), jnp.float32),
            pltpu.VMEM((PERIOD,), jnp.int32),
            pltpu.SemaphoreType.DMA,
        ],
    )
    def k(tab_hbm, idx_hbm, off_hbm, out_hbm, idx_v, rows_v, off_v, sem):
        wid = lax.axis_index("s") * NUM_CORES + lax.axis_index("c")
        pltpu.sync_copy(off_hbm, off_v)

        def body(g, carry):
            base = wid * per_w + g * CHUNK
            row0 = wid * rows_per_w + g * GRP
            pltpu.sync_copy(idx_hbm.at[pl.ds(row0, GRP), :], idx_v)
            for j in range(GRP):
                for t in range(IDX_W // LANES):
                    st = (j * IDX_W + t * LANES) % PERIOD
                    sl = pl.ds(t * LANES, LANES)
                    idx_v[j, sl] = idx_v[j, sl] + off_v[pl.ds(st, LANES)]
            descs = [
                pltpu.async_copy(
                    tab_hbm.at[idx_v.at[j]],
                    rows_v.at[pl.ds(j * IDX_W, IDX_W)],
                    sem,
                )
                for j in range(GRP)
            ]
            for dsc in descs:
                dsc.wait()
            pltpu.sync_copy(rows_v, out_hbm.at[pl.ds(base, CHUNK), :])
            return carry

        lax.fori_loop(0, n_chunks, body, 0)

    return k(tab, idx2, off)


def kernel(input, tables):
    b, t, f = input.shape
    vocab, d = tables.shape[1], tables.shape[2]
    n_rows = b * t * f
    idx2 = input.reshape(n_rows // IDX_W, IDX_W)
    tab = tables.reshape(f * vocab, d)
    off = jnp.tile(jnp.arange(f, dtype=jnp.int32) * vocab, PERIOD // f)
    out = _gather_flat(tab, idx2, off, n_rows, d)
    return out.reshape(b, t, f, d)


# speedup vs baseline: 2.4266x; 2.4266x over previous
import functools

import jax
import jax.numpy as jnp
from jax import lax
from jax.experimental import pallas as pl
from jax.experimental.pallas import tpu as pltpu
from jax.experimental.pallas import tpu_sc as plsc

NUM_CORES = 2
NUM_SUBCORES = 16
NUM_WORKERS = NUM_CORES * NUM_SUBCORES
LANES = 16

IDX_W = 128
CHUNK = 1664
GRP = CHUNK // IDX_W
PERIOD = 208


@functools.partial(jax.jit, static_argnums=(3, 4))
def _gather_flat(tab, idx3, off, n_rows, d):
    per_w = n_rows // NUM_WORKERS
    n_chunks = per_w // CHUNK
    mesh = plsc.VectorSubcoreMesh(core_axis_name="c", subcore_axis_name="s")

    @functools.partial(
        pl.kernel,
        out_type=jax.ShapeDtypeStruct((n_rows, d), jnp.float32),
        mesh=mesh,
        scratch_types=[
            pltpu.VMEM((GRP, IDX_W), jnp.int32),
            pltpu.VMEM((CHUNK, d), jnp.float32),
            pltpu.VMEM((PERIOD,), jnp.int32),
            pltpu.SemaphoreType.DMA,
        ],
        compiler_params=pltpu.CompilerParams(use_tc_tiling_on_sc=False),
    )
    def k(tab_hbm, idx_hbm, off_hbm, out_hbm, idx_v, rows_v, off_v, sem):
        wid = lax.axis_index("s") * NUM_CORES + lax.axis_index("c")
        pltpu.sync_copy(off_hbm, off_v)

        def body(g, carry):
            base = wid * per_w + g * CHUNK
            pltpu.sync_copy(idx_hbm.at[wid * n_chunks + g], idx_v)
            for j in range(GRP):
                for t in range(IDX_W // LANES):
                    st = (j * IDX_W + t * LANES) % PERIOD
                    sl = pl.ds(t * LANES, LANES)
                    idx_v[j, sl] = idx_v[j, sl] + off_v[pl.ds(st, LANES)]
            descs = [
                pltpu.async_copy(
                    tab_hbm.at[idx_v.at[j]],
                    rows_v.at[pl.ds(j * IDX_W, IDX_W)],
                    sem,
                )
                for j in range(GRP)
            ]
            for dsc in descs:
                dsc.wait()
            pltpu.sync_copy(rows_v, out_hbm.at[pl.ds(base, CHUNK), :])
            return carry

        lax.fori_loop(0, n_chunks, body, 0)

    return k(tab, idx3, off)


def kernel(input, tables):
    b, t, f = input.shape
    vocab, d = tables.shape[1], tables.shape[2]
    n_rows = b * t * f
    idx3 = input.reshape(n_rows // CHUNK, GRP, IDX_W)
    tab = tables.reshape(f * vocab, d)
    off = jnp.tile(jnp.arange(f, dtype=jnp.int32) * vocab, PERIOD // f)
    out = _gather_flat(tab, idx3, off, n_rows, d)
    return out.reshape(b, t, f, d)


# speedup vs baseline: 2.4649x; 1.0158x over previous
import functools

import jax
import jax.numpy as jnp
from jax import lax
from jax.experimental import pallas as pl
from jax.experimental.pallas import tpu as pltpu
from jax.experimental.pallas import tpu_sc as plsc

NUM_CORES = 2
NUM_SUBCORES = 16
NUM_WORKERS = NUM_CORES * NUM_SUBCORES
LANES = 16

IDX_W = 128
CHUNK = 1664
GRP = CHUNK // IDX_W
PERIOD = 208


@functools.partial(jax.jit, static_argnums=(3, 4))
def _gather_flat(tab, idx3, off, n_rows, d):
    per_w = n_rows // NUM_WORKERS
    n_chunks = per_w // CHUNK
    mesh = plsc.VectorSubcoreMesh(core_axis_name="c", subcore_axis_name="s")

    @functools.partial(
        pl.kernel,
        out_type=jax.ShapeDtypeStruct((n_rows, d), jnp.float32),
        mesh=mesh,
        scratch_types=[
            pltpu.VMEM((GRP, IDX_W), jnp.int32),
            pltpu.VMEM((GRP, IDX_W), jnp.int32),
            pltpu.VMEM((CHUNK, d), jnp.float32),
            pltpu.VMEM((CHUNK, d), jnp.float32),
            pltpu.VMEM((PERIOD,), jnp.int32),
            pltpu.SemaphoreType.DMA,
            pltpu.SemaphoreType.DMA,
            pltpu.SemaphoreType.DMA,
            pltpu.SemaphoreType.DMA,
        ],
        compiler_params=pltpu.CompilerParams(use_tc_tiling_on_sc=False),
    )
    def k(tab_hbm, idx_hbm, off_hbm, out_hbm,
          idx_a, idx_b, rows_a, rows_b, off_v,
          gsem_a, gsem_b, wsem_a, wsem_b):
        wid = lax.axis_index("s") * NUM_CORES + lax.axis_index("c")
        idx_bufs = (idx_a, idx_b)
        row_bufs = (rows_a, rows_b)
        gsems = (gsem_a, gsem_b)
        wsems = (wsem_a, wsem_b)
        pltpu.sync_copy(off_hbm, off_v)

        def load_idx(g, buf):
            pltpu.sync_copy(idx_hbm.at[wid * n_chunks + g], buf)
            for j in range(GRP):
                for t in range(IDX_W // LANES):
                    st = (j * IDX_W + t * LANES) % PERIOD
                    sl = pl.ds(t * LANES, LANES)
                    buf[j, sl] = buf[j, sl] + off_v[pl.ds(st, LANES)]

        def fire_gathers(ibuf, rbuf, sem):
            return [
                pltpu.async_copy(
                    tab_hbm.at[ibuf.at[j]],
                    rbuf.at[pl.ds(j * IDX_W, IDX_W)],
                    sem,
                )
                for j in range(GRP)
            ]

        def drain_and_write(g, rbuf, sem, descs, wsem):
            for dsc in descs:
                dsc.wait()
            return pltpu.async_copy(
                rbuf, out_hbm.at[pl.ds(wid * per_w + g * CHUNK, CHUNK)], wsem
            )

        def wait_write(b):
            pltpu.make_async_copy(
                row_bufs[b], out_hbm.at[pl.ds(0, CHUNK)], wsems[b]
            ).wait()

        load_idx(0, idx_bufs[0])
        for g in (0, 1):
            b = g % 2
            descs = fire_gathers(idx_bufs[b], row_bufs[b], gsems[b])
            load_idx(g + 1, idx_bufs[1 - b])
            drain_and_write(g, row_bufs[b], gsems[b], descs, wsems[b])

        def body(go, carry):
            for b in (0, 1):
                g = 2 * go + b
                wait_write(b)
                descs = fire_gathers(idx_bufs[b], row_bufs[b], gsems[b])
                load_idx(g + 1, idx_bufs[1 - b])
                drain_and_write(g, row_bufs[b], gsems[b], descs, wsems[b])
            return carry

        lax.fori_loop(1, n_chunks // 2, body, 0)
        wait_write(0)
        wait_write(1)

    return k(tab, idx3, off)


def kernel(input, tables):
    b, t, f = input.shape
    vocab, d = tables.shape[1], tables.shape[2]
    n_rows = b * t * f
    n_chunks_total = n_rows // CHUNK
    idx3 = input.reshape(n_chunks_total, GRP, IDX_W)
    idx3 = jnp.concatenate(
        [idx3, jnp.zeros((1, GRP, IDX_W), jnp.int32)], axis=0
    )
    tab = tables.reshape(f * vocab, d)
    off = jnp.tile(jnp.arange(f, dtype=jnp.int32) * vocab, PERIOD // f)
    out = _gather_flat(tab, idx3, off, n_rows, d)
    return out.reshape(b, t, f, d)
